# baseline (speedup 1.0000x reference)
import jax
import jax.numpy as jnp
from jax import lax
from jax.experimental import pallas as pl
from jax.experimental.pallas import tpu as pltpu
from jax.experimental.pallas import tpu_sc as plsc

VOCAB = 1000000
EMBED_DIM = 64
BATCH = 4096
SEQ = 200

NC = 2
NS = 16
NW = NC * NS

CHUNK = 128
TOTAL = BATCH * SEQ
CPW = TOTAL // (NW * CHUNK)
NBUF = 4


def _body(table_hbm, idx_hbm, out_hbm, idx_v, rows_v, gsem):
  wid = lax.axis_index("s") * NC + lax.axis_index("c")
  row_base = wid * (CPW * CHUNK)

  pltpu.sync_copy(idx_hbm.at[pl.ds(wid * CPW, CPW)], idx_v)

  for b in range(NBUF):
    pltpu.async_copy(table_hbm.at[idx_v.at[b]], rows_v.at[b], gsem.at[b])

  def group(g, carry):
    for b in range(NBUF):
      j = g * NBUF + b
      pltpu.make_async_copy(
          table_hbm.at[idx_v.at[j]], rows_v.at[b], gsem.at[b]).wait()
      pltpu.sync_copy(
          rows_v.at[b], out_hbm.at[pl.ds(row_base + j * CHUNK, CHUNK)])
      jn = j + NBUF
      pltpu.async_copy(table_hbm.at[idx_v.at[jn]], rows_v.at[b], gsem.at[b])
    return carry

  lax.fori_loop(0, CPW // NBUF - 1, group, 0)

  for b in range(NBUF):
    j = (CPW - NBUF) + b
    pltpu.make_async_copy(
        table_hbm.at[idx_v.at[j]], rows_v.at[b], gsem.at[b]).wait()
    pltpu.sync_copy(
        rows_v.at[b], out_hbm.at[pl.ds(row_base + j * CHUNK, CHUNK)])


_gather_cache = []


def _get_gather():
  if not _gather_cache:
    _gather_cache.append(pl.kernel(
        _body,
        out_type=jax.ShapeDtypeStruct((TOTAL, EMBED_DIM), jnp.float32),
        mesh=plsc.VectorSubcoreMesh(
            core_axis_name="c", subcore_axis_name="s",
            num_cores=NC, num_subcores=NS),
        scratch_types=[
            pltpu.VMEM((CPW, CHUNK), jnp.int32),
            pltpu.VMEM((NBUF, CHUNK, EMBED_DIM), jnp.float32),
            pltpu.SemaphoreType.DMA((NBUF,)),
        ],
        compiler_params=pltpu.CompilerParams(use_tc_tiling_on_sc=False),
    ))
  return _gather_cache[0]


@jax.jit
def kernel(words, table):
  idx = words.reshape(TOTAL // CHUNK, CHUNK)
  out = _get_gather()(table, idx)
  return out.reshape(BATCH, SEQ, EMBED_DIM)


# speedup vs baseline: 1.2202x; 1.2202x over previous
import jax
import jax.numpy as jnp
from jax import lax
from jax.experimental import pallas as pl
from jax.experimental.pallas import tpu as pltpu
from jax.experimental.pallas import tpu_sc as plsc

VOCAB = 1000000
EMBED_DIM = 64
BATCH = 4096
SEQ = 200

NC = 2
NS = 16
NW = NC * NS

CHUNK = 128
TOTAL = BATCH * SEQ
CPW = TOTAL // (NW * CHUNK)
NBUF = 4
ROWW = 2 * EMBED_DIM


def _body(tab_hbm, idx_hbm, out_hbm, idx_v, rows_v, gsem):
  wid = lax.axis_index("s") * NC + lax.axis_index("c")
  row_base = wid * (CPW * CHUNK)

  pltpu.sync_copy(idx_hbm.at[pl.ds(wid * CPW, CPW)], idx_v)

  for b in range(NBUF):
    pltpu.async_copy(tab_hbm.at[idx_v.at[b]], rows_v.at[b], gsem.at[b])

  def group(g, carry):
    for b in range(NBUF):
      j = g * NBUF + b
      pltpu.make_async_copy(
          tab_hbm.at[idx_v.at[j]], rows_v.at[b], gsem.at[b]).wait()
      pltpu.sync_copy(
          rows_v.at[b], out_hbm.at[pl.ds(row_base + j * CHUNK, CHUNK)])
      jn = j + NBUF
      pltpu.async_copy(tab_hbm.at[idx_v.at[jn]], rows_v.at[b], gsem.at[b])
    return carry

  lax.fori_loop(0, CPW // NBUF - 1, group, 0)

  for b in range(NBUF):
    j = (CPW - NBUF) + b
    pltpu.make_async_copy(
        tab_hbm.at[idx_v.at[j]], rows_v.at[b], gsem.at[b]).wait()
    pltpu.sync_copy(
        rows_v.at[b], out_hbm.at[pl.ds(row_base + j * CHUNK, CHUNK)])


_gather_cache = []


def _get_gather():
  if not _gather_cache:
    _gather_cache.append(pl.kernel(
        _body,
        out_type=jax.ShapeDtypeStruct((TOTAL, ROWW), jnp.float32),
        mesh=plsc.VectorSubcoreMesh(
            core_axis_name="c", subcore_axis_name="s",
            num_cores=NC, num_subcores=NS),
        scratch_types=[
            pltpu.VMEM((CPW, CHUNK), jnp.int32),
            pltpu.VMEM((NBUF, CHUNK, ROWW), jnp.float32),
            pltpu.SemaphoreType.DMA((NBUF,)),
        ],
        compiler_params=pltpu.CompilerParams(use_tc_tiling_on_sc=False),
    ))
  return _gather_cache[0]


@jax.jit
def kernel(words, table):
  idx = words.reshape(TOTAL // CHUNK, CHUNK)
  tab = jnp.pad(table, ((0, 0), (0, ROWW - EMBED_DIM)))
  wide = _get_gather()(tab, idx)
  return wide[:, :EMBED_DIM].reshape(BATCH, SEQ, EMBED_DIM)
